# Initial kernel scaffold; baseline (speedup 1.0000x reference)
import jax
import jax.numpy as jnp
from jax.experimental import pallas as pl


def kernel(feats, patch_ids, num_patches, W1, b1, W2, b2):
    raise NotImplementedError("write your pallas kernel here")


# speedup vs baseline: 1.9236x; 1.9236x over previous
import functools

import jax
import jax.numpy as jnp
from jax import lax
from jax.experimental import pallas as pl
from jax.experimental.pallas import tpu as pltpu
from jax.experimental.pallas import tpu_sc as plsc


def _sc_gather(flat, patch_ids):
    B, C, HW = flat.shape
    K = patch_ids.shape[1]
    info = plsc.get_sparse_core_info()
    NC, NS, L = info.num_cores, info.num_subcores, info.num_lanes
    NW = NC * NS
    assert NW % B == 0
    WPB = NW // B
    CPW = C // WPB
    assert CPW * WPB == C
    mesh = plsc.VectorSubcoreMesh(core_axis_name="c", subcore_axis_name="s")

    @functools.partial(
        pl.kernel,
        mesh=mesh,
        out_type=jax.ShapeDtypeStruct((B, C, K), jnp.float32),
        scratch_types=[
            pltpu.VMEM((K,), jnp.int32),
            pltpu.VMEM((HW,), jnp.float32),
            pltpu.VMEM((K,), jnp.float32),
        ],
        compiler_params=pltpu.CompilerParams(needs_layout_passes=False),
    )
    def gather_kernel(flat_hbm, ids_hbm, out_hbm, ids_v, row_v, out_v):
        wid = lax.axis_index("s") * NC + lax.axis_index("c")
        b = wid // WPB
        c0 = (wid % WPB) * CPW
        pltpu.sync_copy(ids_hbm.at[b], ids_v)

        def chan_body(ci, _):
            c = c0 + ci
            pltpu.sync_copy(flat_hbm.at[b, c], row_v)

            def gat_body(j, _):
                idx = ids_v[pl.ds(j * L, L)]
                out_v[pl.ds(j * L, L)] = plsc.load_gather(row_v, [idx])
                return 0

            lax.fori_loop(0, K // L, gat_body, 0)
            pltpu.sync_copy(out_v, out_hbm.at[b, c])
            return 0

        lax.fori_loop(0, CPW, chan_body, 0)

    return gather_kernel(flat, patch_ids)


def _mlp_body(g_ref, w1_ref, b1_ref, w2_ref, b2_ref, out_ref):
    g = g_ref[0]
    h = lax.dot_general(g, w1_ref[...], (((0,), (0,)), ((), ())),
                        preferred_element_type=jnp.float32)
    h = jnp.maximum(h + b1_ref[...], 0.0)
    p = jnp.dot(h, w2_ref[...], preferred_element_type=jnp.float32)
    p = p + b2_ref[...]
    nrm = jnp.sqrt(jnp.sum(p * p, axis=1, keepdims=True))
    out_ref[...] = p / jnp.maximum(nrm, 1e-12)


def _tc_mlp(gT, W1, b1, W2, b2, interpret=False):
    B, C, K = gT.shape
    P = W1.shape[1]
    KC = K
    grid = (B, K // KC)

    return pl.pallas_call(
        _mlp_body,
        grid=grid,
        in_specs=[
            pl.BlockSpec((1, C, KC), lambda b, k: (b, 0, k)),
            pl.BlockSpec((C, P), lambda b, k: (0, 0)),
            pl.BlockSpec((1, P), lambda b, k: (0, 0)),
            pl.BlockSpec((P, P), lambda b, k: (0, 0)),
            pl.BlockSpec((1, P), lambda b, k: (0, 0)),
        ],
        out_specs=pl.BlockSpec((KC, P), lambda b, k: (b * (K // KC) + k, 0)),
        out_shape=jax.ShapeDtypeStruct((B * K, P), jnp.float32),
        interpret=interpret,
    )(gT, W1, b1.reshape(1, P), W2, b2.reshape(1, P))


def kernel(feats, patch_ids, num_patches, W1, b1, W2, b2):
    B, C, H, W = feats.shape
    flat = feats.reshape(B, C, H * W)
    gT = _sc_gather(flat, patch_ids)
    p = _tc_mlp(gT, W1, b1, W2, b2)
    return (p, patch_ids)


# speedup vs baseline: 2.3091x; 1.2004x over previous
import functools

import jax
import jax.numpy as jnp
from jax import lax
from jax.experimental import pallas as pl
from jax.experimental.pallas import tpu as pltpu
from jax.experimental.pallas import tpu_sc as plsc


def _sc_gather(flat, patch_ids):
    B, C, HW = flat.shape
    K = patch_ids.shape[1]
    info = plsc.get_sparse_core_info()
    NC, NS, L = info.num_cores, info.num_subcores, info.num_lanes
    NW = NC * NS
    assert NW % B == 0
    WPB = NW // B
    CPW = C // WPB
    assert CPW * WPB == C
    mesh = plsc.VectorSubcoreMesh(core_axis_name="c", subcore_axis_name="s")

    assert CPW % 2 == 0

    @functools.partial(
        pl.kernel,
        mesh=mesh,
        out_type=jax.ShapeDtypeStruct((B, C, K), jnp.float32),
        scratch_types=[
            pltpu.VMEM((K,), jnp.int32),
            pltpu.VMEM((HW,), jnp.float32),
            pltpu.VMEM((HW,), jnp.float32),
            pltpu.VMEM((K,), jnp.float32),
            pltpu.VMEM((K,), jnp.float32),
            pltpu.SemaphoreType.DMA,
            pltpu.SemaphoreType.DMA,
        ],
        compiler_params=pltpu.CompilerParams(needs_layout_passes=False),
    )
    def gather_kernel(flat_hbm, ids_hbm, out_hbm, ids_v, row0_v, row1_v,
                      out0_v, out1_v, sem_in, sem_out):
        wid = lax.axis_index("s") * NC + lax.axis_index("c")
        b = wid // WPB
        c0 = (wid % WPB) * CPW
        pltpu.sync_copy(ids_hbm.at[b], ids_v)
        pltpu.async_copy(flat_hbm.at[b, c0], row0_v, sem_in)

        def gather_row(row, other_row, ob, c):
            pltpu.make_async_copy(flat_hbm.at[b, c], row, sem_in).wait()

            @pl.when(c + 1 < c0 + CPW)
            def _():
                pltpu.async_copy(flat_hbm.at[b, c + 1], other_row, sem_in)

            @pl.when(c - 2 >= c0)
            def _():
                pltpu.make_async_copy(ob, out_hbm.at[b, c], sem_out).wait()

            for j in range(K // L):
                idx = ids_v[pl.ds(j * L, L)]
                ob[pl.ds(j * L, L)] = plsc.load_gather(row, [idx])
            pltpu.async_copy(ob, out_hbm.at[b, c], sem_out)

        def chan_body(ci2, _):
            c = c0 + ci2 * 2
            gather_row(row0_v, row1_v, out0_v, c)
            gather_row(row1_v, row0_v, out1_v, c + 1)
            return 0

        lax.fori_loop(0, CPW // 2, chan_body, 0)
        pltpu.make_async_copy(out0_v, out_hbm.at[b, c0], sem_out).wait()
        pltpu.make_async_copy(out1_v, out_hbm.at[b, c0], sem_out).wait()

    return gather_kernel(flat, patch_ids)


def _mlp_body(g_ref, w1_ref, b1_ref, w2_ref, b2_ref, out_ref):
    g = g_ref[0]
    h = lax.dot_general(g, w1_ref[...], (((0,), (0,)), ((), ())),
                        preferred_element_type=jnp.float32)
    h = jnp.maximum(h + b1_ref[...], 0.0)
    p = jnp.dot(h, w2_ref[...], preferred_element_type=jnp.float32)
    p = p + b2_ref[...]
    nrm = jnp.sqrt(jnp.sum(p * p, axis=1, keepdims=True))
    out_ref[...] = p / jnp.maximum(nrm, 1e-12)


def _tc_mlp(gT, W1, b1, W2, b2, interpret=False):
    B, C, K = gT.shape
    P = W1.shape[1]
    KC = K
    grid = (B, K // KC)

    return pl.pallas_call(
        _mlp_body,
        grid=grid,
        in_specs=[
            pl.BlockSpec((1, C, KC), lambda b, k: (b, 0, k)),
            pl.BlockSpec((C, P), lambda b, k: (0, 0)),
            pl.BlockSpec((1, P), lambda b, k: (0, 0)),
            pl.BlockSpec((P, P), lambda b, k: (0, 0)),
            pl.BlockSpec((1, P), lambda b, k: (0, 0)),
        ],
        out_specs=pl.BlockSpec((KC, P), lambda b, k: (b * (K // KC) + k, 0)),
        out_shape=jax.ShapeDtypeStruct((B * K, P), jnp.float32),
        interpret=interpret,
    )(gT, W1, b1.reshape(1, P), W2, b2.reshape(1, P))


def kernel(feats, patch_ids, num_patches, W1, b1, W2, b2):
    B, C, H, W = feats.shape
    flat = feats.reshape(B, C, H * W)
    gT = _sc_gather(flat, patch_ids)
    p = _tc_mlp(gT, W1, b1, W2, b2)
    return (p, patch_ids)


# speedup vs baseline: 4.4386x; 1.9222x over previous
import functools

import jax
import jax.numpy as jnp
from jax import lax
from jax.experimental import pallas as pl
from jax.experimental.pallas import tpu as pltpu
from jax.experimental.pallas import tpu_sc as plsc


def _sc_gather(feats, ids3):
    B, C, H, W = feats.shape
    KH = ids3.shape[1]
    K = KH * 128
    info = plsc.get_sparse_core_info()
    NC, NS, L = info.num_cores, info.num_subcores, info.num_lanes
    NW = NC * NS
    assert NW % B == 0
    WPB = NW // B
    CPW = C // WPB
    assert CPW * WPB == C and CPW % 2 == 0
    mesh = plsc.VectorSubcoreMesh(core_axis_name="c", subcore_axis_name="s")

    @functools.partial(
        pl.kernel,
        mesh=mesh,
        out_type=jax.ShapeDtypeStruct((B, C, KH, 128), jnp.float32),
        scratch_types=[
            pltpu.VMEM((KH, 128), jnp.int32),
            pltpu.VMEM((H, W), jnp.float32),
            pltpu.VMEM((H, W), jnp.float32),
            pltpu.VMEM((KH, 128), jnp.float32),
            pltpu.VMEM((KH, 128), jnp.float32),
            pltpu.SemaphoreType.DMA,
            pltpu.SemaphoreType.DMA,
        ],
        compiler_params=pltpu.CompilerParams(
            needs_layout_passes=False, use_tc_tiling_on_sc=True),
    )
    def gather_kernel(feats_hbm, ids_hbm, out_hbm, ids_v, row0_v, row1_v,
                      out0_v, out1_v, sem_in, sem_out):
        wid = lax.axis_index("s") * NC + lax.axis_index("c")
        b = wid // WPB
        c0 = (wid % WPB) * CPW
        pltpu.sync_copy(ids_hbm.at[b], ids_v)
        pltpu.async_copy(feats_hbm.at[b, c0], row0_v, sem_in)

        def gather_row(row, other_row, ob, c):
            pltpu.make_async_copy(feats_hbm.at[b, c], row, sem_in).wait()

            @pl.when(c + 1 < c0 + CPW)
            def _():
                pltpu.async_copy(feats_hbm.at[b, c + 1], other_row, sem_in)

            @pl.when(c - 2 >= c0)
            def _():
                pltpu.make_async_copy(ob, out_hbm.at[b, c], sem_out).wait()

            G = 8
            for j0 in range(0, K // L, G):
                idxs = [ids_v[(j0 + t) // 8, pl.ds(((j0 + t) % 8) * L, L)]
                        for t in range(G)]
                vals = [plsc.load_gather(
                            row, [jnp.right_shift(ix, 7),
                                  jnp.bitwise_and(ix, 127)])
                        for ix in idxs]
                for t in range(G):
                    j = j0 + t
                    ob[j // 8, pl.ds((j % 8) * L, L)] = vals[t]
            pltpu.async_copy(ob, out_hbm.at[b, c], sem_out)

        def chan_body(ci2, _):
            c = c0 + ci2 * 2
            gather_row(row0_v, row1_v, out0_v, c)
            gather_row(row1_v, row0_v, out1_v, c + 1)
            return 0

        lax.fori_loop(0, CPW // 2, chan_body, 0)
        pltpu.make_async_copy(out0_v, out_hbm.at[b, c0], sem_out).wait()
        pltpu.make_async_copy(out1_v, out_hbm.at[b, c0], sem_out).wait()

    return gather_kernel(feats, ids3)


def _make_mlp_body(KH):
    def _mlp_body(g_ref, w1_ref, b1_ref, w2_ref, b2_ref, out_ref):
        for k in range(KH):
            g = g_ref[0, :, k, :]
            h = lax.dot_general(g, w1_ref[...], (((0,), (0,)), ((), ())),
                                preferred_element_type=jnp.float32)
            h = jnp.maximum(h + b1_ref[...], 0.0)
            p = jnp.dot(h, w2_ref[...], preferred_element_type=jnp.float32)
            p = p + b2_ref[...]
            nrm = jnp.sqrt(jnp.sum(p * p, axis=1, keepdims=True))
            out_ref[pl.ds(k * 128, 128), :] = p / jnp.maximum(nrm, 1e-12)
    return _mlp_body


def _tc_mlp(gT, W1, b1, W2, b2, interpret=False):
    B, C, KH, _ = gT.shape
    P = W1.shape[1]

    return pl.pallas_call(
        _make_mlp_body(KH),
        grid=(B,),
        in_specs=[
            pl.BlockSpec((1, C, KH, 128), lambda b: (b, 0, 0, 0)),
            pl.BlockSpec((C, P), lambda b: (0, 0)),
            pl.BlockSpec((1, P), lambda b: (0, 0)),
            pl.BlockSpec((P, P), lambda b: (0, 0)),
            pl.BlockSpec((1, P), lambda b: (0, 0)),
        ],
        out_specs=pl.BlockSpec((KH * 128, P), lambda b: (b, 0)),
        out_shape=jax.ShapeDtypeStruct((B * KH * 128, P), jnp.float32),
        interpret=interpret,
    )(gT, W1, b1.reshape(1, P), W2, b2.reshape(1, P))


def kernel(feats, patch_ids, num_patches, W1, b1, W2, b2):
    B, C, H, W = feats.shape
    K = patch_ids.shape[1]
    ids3 = patch_ids.reshape(B, K // 128, 128)
    gT = _sc_gather(feats, ids3)
    p = _tc_mlp(gT, W1, b1, W2, b2)
    return (p, patch_ids)


# speedup vs baseline: 5.6690x; 1.2772x over previous
import functools

import jax
import jax.numpy as jnp
from jax import lax
from jax.experimental import pallas as pl
from jax.experimental.pallas import tpu as pltpu
from jax.experimental.pallas import tpu_sc as plsc

_NBUF = 3


def _sc_gather(feats, ids3):
    B, C, H, W = feats.shape
    KH = ids3.shape[1]
    K = KH * 128
    info = plsc.get_sparse_core_info()
    NC, NS, L = info.num_cores, info.num_subcores, info.num_lanes
    NW = NC * NS
    assert NW % B == 0
    WPB = NW // B
    CPW = C // WPB
    assert CPW * WPB == C and CPW % _NBUF == 0
    mesh = plsc.VectorSubcoreMesh(core_axis_name="c", subcore_axis_name="s")

    @functools.partial(
        pl.kernel,
        mesh=mesh,
        out_type=jax.ShapeDtypeStruct((B, C, K), jnp.float32),
        scratch_types=[
            pltpu.VMEM((KH, 128), jnp.int32),
            [pltpu.VMEM((H, W), jnp.float32) for _ in range(_NBUF)],
            [pltpu.VMEM((K,), jnp.float32) for _ in range(_NBUF)],
            [pltpu.SemaphoreType.DMA for _ in range(_NBUF)],
            [pltpu.SemaphoreType.DMA for _ in range(_NBUF)],
        ],
        compiler_params=pltpu.CompilerParams(
            needs_layout_passes=False, use_tc_tiling_on_sc=True),
    )
    def gather_kernel(feats_hbm, ids_hbm, out_hbm, ids_v, rows, outs,
                      sems_in, sems_out):
        wid = lax.axis_index("s") * NC + lax.axis_index("c")
        b = wid // WPB
        c0 = (wid % WPB) * CPW
        pltpu.sync_copy(ids_hbm.at[b], ids_v)
        for p in range(_NBUF - 1):
            pltpu.async_copy(feats_hbm.at[b, c0 + p], rows[p], sems_in[p])

        def gather_row(p, c):
            pltpu.make_async_copy(feats_hbm.at[b, c], rows[p],
                                  sems_in[p]).wait()

            @pl.when(c + _NBUF - 1 < c0 + CPW)
            def _():
                pltpu.async_copy(feats_hbm.at[b, c + _NBUF - 1],
                                 rows[(p + _NBUF - 1) % _NBUF],
                                 sems_in[(p + _NBUF - 1) % _NBUF])

            ob = outs[p]

            @pl.when(c - _NBUF >= c0)
            def _():
                pltpu.make_async_copy(ob, out_hbm.at[b, c], sems_out[p]).wait()

            row = rows[p]
            G = 8
            for j0 in range(0, K // L, G):
                idxs = [ids_v[(j0 + t) // 8, pl.ds(((j0 + t) % 8) * L, L)]
                        for t in range(G)]
                vals = [plsc.load_gather(
                            row, [jnp.right_shift(ix, 7),
                                  jnp.bitwise_and(ix, 127)])
                        for ix in idxs]
                for t in range(G):
                    ob[pl.ds((j0 + t) * L, L)] = vals[t]
            pltpu.async_copy(ob, out_hbm.at[b, c], sems_out[p])

        def chan_body(ci, _):
            c = c0 + ci * _NBUF
            for p in range(_NBUF):
                gather_row(p, c + p)
            return 0

        lax.fori_loop(0, CPW // _NBUF, chan_body, 0)
        for p in range(_NBUF):
            pltpu.make_async_copy(outs[p], out_hbm.at[b, c0],
                                  sems_out[p]).wait()

    return gather_kernel(feats, ids3)


def _mlp_body(g_ref, w1_ref, b1_ref, w2_ref, b2_ref, out_ref):
    g = g_ref[0]
    h = lax.dot_general(g, w1_ref[...], (((0,), (0,)), ((), ())),
                        preferred_element_type=jnp.float32)
    h = jnp.maximum(h + b1_ref[...], 0.0)
    p = jnp.dot(h, w2_ref[...], preferred_element_type=jnp.float32)
    p = p + b2_ref[...]
    nrm = jnp.sqrt(jnp.sum(p * p, axis=1, keepdims=True))
    out_ref[...] = p / jnp.maximum(nrm, 1e-12)


def _tc_mlp(gT, W1, b1, W2, b2, interpret=False):
    B, C, K = gT.shape
    P = W1.shape[1]

    return pl.pallas_call(
        _mlp_body,
        grid=(B,),
        in_specs=[
            pl.BlockSpec((1, C, K), lambda b: (b, 0, 0)),
            pl.BlockSpec((C, P), lambda b: (0, 0)),
            pl.BlockSpec((1, P), lambda b: (0, 0)),
            pl.BlockSpec((P, P), lambda b: (0, 0)),
            pl.BlockSpec((1, P), lambda b: (0, 0)),
        ],
        out_specs=pl.BlockSpec((K, P), lambda b: (b, 0)),
        out_shape=jax.ShapeDtypeStruct((B * K, P), jnp.float32),
        interpret=interpret,
    )(gT, W1, b1.reshape(1, P), W2, b2.reshape(1, P))


def kernel(feats, patch_ids, num_patches, W1, b1, W2, b2):
    B, C, H, W = feats.shape
    K = patch_ids.shape[1]
    ids3 = patch_ids.reshape(B, K // 128, 128)
    gT = _sc_gather(feats, ids3)
    p = _tc_mlp(gT, W1, b1, W2, b2)
    return (p, patch_ids)
